# baseline (device time: 170709 ns/iter reference)
import jax
import jax.numpy as jnp
from jax import lax
from jax.experimental import pallas as pl
from jax.experimental.pallas import tpu as pltpu

N_DEV = 4


def kernel(x, w_mat, scale_x, scale_w):
    m, k_per = x.shape
    _, n = w_mat.shape
    mc = m // N_DEV
    h = n // 2
    hh = h // 2

    def body(x_ref, w_ref, sx_ref, sw_ref, out_ref, acc_ref,
             comm_r_ref, comm_l_ref, w8_ref,
             send_r, recv_r, send_l, recv_l, copy_sems):
        my_pos = lax.axis_index("i")
        left = lax.rem(my_pos - 1 + N_DEV, N_DEV)
        right = lax.rem(my_pos + 1, N_DEV)

        def rows_of(c):
            return pl.ds(lax.rem(c + 2 * N_DEV, N_DEV) * mc, mc)

        barrier_sem = pltpu.get_barrier_semaphore()
        for nbr in (left, right):
            pl.semaphore_signal(
                barrier_sem, inc=1,
                device_id=(nbr,), device_id_type=pl.DeviceIdType.MESH,
            )
        pl.semaphore_wait(barrier_sem, 2)

        def gemm(c, col0, width):
            rows = rows_of(c)
            cols = pl.ds(col0, width)
            acc_ref[rows, cols] = jnp.dot(
                x_ref[rows, :].astype(jnp.float8_e4m3fn), w8_ref[:, cols],
                preferred_element_type=jnp.float32,
            ).astype(jnp.bfloat16)

        def rs_rdma(s, sub, dirn):
            slot = s % 2
            if dirn == 0:
                src = acc_ref.at[rows_of(my_pos - s), pl.ds(sub * hh, hh)]
                dst = comm_r_ref.at[slot, :, pl.ds(sub * hh, hh)]
                return pltpu.make_async_remote_copy(
                    src_ref=src, dst_ref=dst,
                    send_sem=send_r.at[2 * s + sub],
                    recv_sem=recv_r.at[2 * s + sub],
                    device_id=(right,), device_id_type=pl.DeviceIdType.MESH)
            src = acc_ref.at[rows_of(my_pos + s), pl.ds(h + sub * hh, hh)]
            dst = comm_l_ref.at[slot, :, pl.ds(sub * hh, hh)]
            return pltpu.make_async_remote_copy(
                src_ref=src, dst_ref=dst,
                send_sem=send_l.at[2 * s + sub],
                recv_sem=recv_l.at[2 * s + sub],
                device_id=(left,), device_id_type=pl.DeviceIdType.MESH)

        def ag_rdma(t, sub, dirn):
            if dirn == 0:
                ref = acc_ref.at[rows_of(my_pos + 1 - t), pl.ds(sub * hh, hh)]
                return pltpu.make_async_remote_copy(
                    src_ref=ref, dst_ref=ref,
                    send_sem=send_r.at[6 + 2 * t + sub],
                    recv_sem=recv_r.at[6 + 2 * t + sub],
                    device_id=(right,), device_id_type=pl.DeviceIdType.MESH)
            ref = acc_ref.at[rows_of(my_pos - 1 + t), pl.ds(h + sub * hh, hh)]
            return pltpu.make_async_remote_copy(
                src_ref=ref, dst_ref=ref,
                send_sem=send_l.at[6 + 2 * t + sub],
                recv_sem=recv_l.at[6 + 2 * t + sub],
                device_id=(left,), device_id_type=pl.DeviceIdType.MESH)

        inflight = {}
        for dirn in (0, 1):
            wcols = pl.ds(dirn * h, h)
            w8_ref[:, wcols] = w_ref[:, wcols].astype(jnp.float8_e5m2)
            for sub in (0, 1):
                gemm(my_pos, dirn * h + sub * hh, hh)
                d = rs_rdma(0, sub, dirn)
                d.start()
                inflight[(0, sub, dirn)] = d

        for s in range(N_DEV - 2):
            slot = s % 2
            if s == 0:
                gemm(my_pos - 1, 0, h)
                gemm(my_pos + 1, h, h)
            else:
                gemm(my_pos + 2, 0, n)
            for sub in (0, 1):
                for dirn in (0, 1):
                    inflight.pop((s, sub, dirn)).wait()
                    if dirn == 0:
                        rr = rows_of(my_pos - s - 1)
                        cols = pl.ds(sub * hh, hh)
                        acc_ref[rr, cols] = (
                            acc_ref[rr, cols]
                            + comm_r_ref[slot, :, pl.ds(sub * hh, hh)])
                    else:
                        rl = rows_of(my_pos + s + 1)
                        cols = pl.ds(h + sub * hh, hh)
                        acc_ref[rl, cols] = (
                            acc_ref[rl, cols]
                            + comm_l_ref[slot, :, pl.ds(sub * hh, hh)])
                    nxt = rs_rdma(s + 1, sub, dirn)
                    nxt.start()
                    inflight[(s + 1, sub, dirn)] = nxt

        gemm(my_pos + 1, 0, h)
        gemm(my_pos - 1, h, h)

        n_stores = [0]
        pending = {}

        def store_sub(c, col0):
            k = n_stores[0]
            n_stores[0] += 1
            slot = k % 4
            if slot in pending:
                pending[slot].wait()
            rows = rows_of(c)
            cols = pl.ds(col0, hh)
            cp = pltpu.make_async_copy(
                acc_ref.at[rows, cols], out_ref.at[rows, cols],
                copy_sems.at[slot]
            )
            cp.start()
            pending[slot] = cp

        scale = sx_ref[0] * sw_ref[0]
        for sub in (0, 1):
            for dirn in (0, 1):
                inflight.pop((2, sub, dirn)).wait()
                if dirn == 0:
                    rr = rows_of(my_pos + 1)
                    cols = pl.ds(sub * hh, hh)
                    acc_ref[rr, cols] = jnp.maximum(
                        (acc_ref[rr, cols].astype(jnp.float32)
                         + comm_r_ref[0, :, pl.ds(sub * hh, hh)].astype(
                             jnp.float32)) * scale, 0.0
                    ).astype(jnp.bfloat16)
                else:
                    rl = rows_of(my_pos - 1)
                    cols = pl.ds(h + sub * hh, hh)
                    acc_ref[rl, cols] = jnp.maximum(
                        (acc_ref[rl, cols].astype(jnp.float32)
                         + comm_l_ref[0, :, pl.ds(sub * hh, hh)].astype(
                             jnp.float32)) * scale, 0.0
                    ).astype(jnp.bfloat16)
                d = ag_rdma(0, sub, dirn)
                d.start()
                inflight[(0, sub, dirn)] = d
            store_sub(my_pos + 1, sub * hh)
            store_sub(my_pos - 1, h + sub * hh)

        for t in range(N_DEV - 1):
            for sub in (0, 1):
                for dirn in (0, 1):
                    inflight.pop((t, sub, dirn)).wait()
                    if t < N_DEV - 2:
                        nxt = ag_rdma(t + 1, sub, dirn)
                        nxt.start()
                        inflight[(t + 1, sub, dirn)] = nxt
                store_sub(my_pos - t, sub * hh)
                store_sub(my_pos + t, h + sub * hh)

        for cp in pending.values():
            cp.wait()

    out = pl.pallas_call(
        body,
        out_shape=jax.ShapeDtypeStruct((m, n), jnp.bfloat16),
        in_specs=[
            pl.BlockSpec(memory_space=pltpu.VMEM),
            pl.BlockSpec(memory_space=pltpu.VMEM),
            pl.BlockSpec(memory_space=pltpu.SMEM),
            pl.BlockSpec(memory_space=pltpu.SMEM),
        ],
        out_specs=pl.BlockSpec(memory_space=pl.ANY),
        scratch_shapes=[
            pltpu.VMEM((m, n), jnp.bfloat16),
            pltpu.VMEM((2, mc, h), jnp.bfloat16),
            pltpu.VMEM((2, mc, h), jnp.bfloat16),
            pltpu.VMEM((k_per, n), jnp.float8_e5m2),
            pltpu.SemaphoreType.DMA((12,)),
            pltpu.SemaphoreType.DMA((12,)),
            pltpu.SemaphoreType.DMA((12,)),
            pltpu.SemaphoreType.DMA((12,)),
            pltpu.SemaphoreType.DMA((4,)),
        ],
        compiler_params=pltpu.CompilerParams(
            collective_id=0, vmem_limit_bytes=100 * 1024 * 1024),
    )(x, w_mat, scale_x, scale_w)
    return out.astype(jnp.float32)
